# baseline (speedup 1.0000x reference)
import functools

import jax
import jax.numpy as jnp
from jax import lax
from jax.experimental import pallas as pl
from jax.experimental.pallas import tpu as pltpu
from jax.experimental.pallas import tpu_sc as plsc

_G1 = 3.0
_G2 = 3.0
_PSHIFT = 15.0
_NPAD = 640
_NC = 2
_NS = 16
_CHUNKS = 4
_CVECS = _NPAD // (_CHUNKS * 16)
_CPAD = _NPAD + 32


def _prep_kernel(lc_ref, cls_ref, ll_ref, pb_ref, lt_ref, pr_ref):
    mask = lc_ref[:, 0, :] > 0
    p = jnp.exp(cls_ref[:, 1, :])

    bx1 = pb_ref[:, 0, :]
    by1 = pb_ref[:, 1, :]
    bx2 = pb_ref[:, 2, :]
    by2 = pb_ref[:, 3, :]
    tx1 = lt_ref[:, 0:1]
    ty1 = lt_ref[:, 1:2]
    tx2 = lt_ref[:, 2:3]
    ty2 = lt_ref[:, 3:4]

    xx1 = jnp.maximum(tx1, bx1)
    yy1 = jnp.maximum(ty1, by1)
    xx2 = jnp.minimum(tx2, bx2)
    yy2 = jnp.minimum(ty2, by2)
    ww = jnp.maximum(xx2 - xx1, 0.0)
    hh = jnp.maximum(yy2 - yy1, 0.0)
    area = (bx2 - bx1) * (by2 - by1)
    ta = (tx2 - tx1) * (ty2 - ty1)
    inter = ww * hh
    iou = inter / (area + ta - inter)

    cx = ll_ref[:, 0, :] + tx1
    cy = ll_ref[:, 1, :] + ty1
    tcx = (tx1 + tx2) / 2.0
    tcy = (ty1 + ty2) / 2.0
    dist = jnp.sqrt((cx - tcx) ** 2 + (cy - tcy) ** 2)

    ps = p - _PSHIFT
    pr_ref[:, 0, :] = jnp.where(mask, dist, -1e30)
    pr_ref[:, 1, :] = jnp.exp(_G1 * ps)
    pr_ref[:, 2, :] = jnp.exp(_G2 * iou)
    pr_ref[:, 3, :] = jnp.where(mask, dist + 1.0, 1e30)
    pr_ref[:, 4, :] = jnp.where(mask, jnp.exp(-_G1 * ps), 0.0)
    pr_ref[:, 5, :] = jnp.where(mask, jnp.exp(-_G2 * iou), 0.0)


def _sc_pair_kernel(pr_hbm, out_hbm, buf, obuf):
    sample = lax.axis_index("s")
    half = lax.axis_index("c")

    pltpu.sync_copy(pr_hbm.at[sample], buf)

    zero = jnp.zeros((16,), jnp.float32)
    one = jnp.ones((16,), jnp.float32)

    abase = half * (_NPAD // 2)
    t1, t2, t3 = zero, zero, zero
    for c in range(_CHUNKS):
        bd = [buf[0, pl.ds(c * _CVECS * 16 + j * 16, 16)] for j in range(_CVECS)]
        bp = [buf[1, pl.ds(c * _CVECS * 16 + j * 16, 16)] for j in range(_CVECS)]
        bi = [buf[2, pl.ds(c * _CVECS * 16 + j * 16, 16)] for j in range(_CVECS)]

        def body(k, carry, bd=bd, bp=bp, bi=bi):
            t1, t2, t3 = carry
            a0 = abase + k * 16
            tav = buf[3, pl.ds(a0, 16)]
            e1pv = buf[4, pl.ds(a0, 16)]
            e1iv = buf[5, pl.ds(a0, 16)]
            for l in range(16):
                ta = tav[l]
                v1 = v2 = v3 = zero
                for j in range(_CVECS):
                    mf = jnp.where(bd[j] >= ta, one, zero)
                    v1 = v1 + mf * bp[j]
                    v2 = v2 + mf * bi[j]
                    v3 = v3 + mf
                t1 = t1 + e1pv[l] * v1
                t2 = t2 + e1iv[l] * v2
                t3 = t3 + v3
            return (t1, t2, t3)

        t1, t2, t3 = lax.fori_loop(0, _NPAD // 32, body, (t1, t2, t3))

    obuf[pl.ds(0, 16)] = t1
    obuf[pl.ds(16, 16)] = t2
    obuf[pl.ds(32, 16)] = t3
    pltpu.sync_copy(obuf, out_hbm.at[half * _NS + sample])


def _finalize_kernel(parts_ref, did_ref, l1_ref, l2_ref):
    s = parts_ref[0:_NS, :] + parts_ref[_NS:2 * _NS, :]
    s1 = jnp.sum(s[:, 0:16], axis=1, keepdims=True)
    s2 = jnp.sum(s[:, 16:32], axis=1, keepdims=True)
    cnt = jnp.sum(s[:, 32:48], axis=1, keepdims=True)
    did = did_ref[:, 0, :]
    valid = (did != 1) & (cnt > 0.0)
    vf = valid.astype(jnp.float32)
    l1 = jnp.where(valid, s1 / cnt, 0.0)
    l2 = jnp.where(valid, s2 / cnt, 0.0)
    nv = jnp.sum(vf)
    l1_ref[0, 0] = jnp.where(nv > 0.0, jnp.sum(l1) / nv, 0.0)
    l2_ref[0, 0] = jnp.where(nv > 0.0, jnp.sum(l2) / nv, 0.0)


def kernel(cls, label_cls, label_loc, pred_bboxes, label_target, dataset_id):
    B = label_cls.shape[0]
    N = label_cls.shape[2] * label_cls.shape[3]
    assert B == _NS and N <= _NPAD
    pad = _NPAD - N

    lc = jnp.pad(jnp.reshape(label_cls, (B, 1, N)), ((0, 0), (0, 0), (0, pad)))
    cls_t = jnp.pad(jnp.transpose(jnp.reshape(cls, (B, N, 2)), (0, 2, 1)),
                    ((0, 0), (0, 0), (0, pad)))
    ll = jnp.pad(jnp.reshape(label_loc, (B, 4, N)), ((0, 0), (0, 0), (0, pad)))
    pb = jnp.pad(pred_bboxes, ((0, 0), (0, 0), (0, pad)))
    lt = jnp.reshape(label_target, (B, 4))
    did = jnp.reshape(dataset_id, (B, 1, 1))

    pr = pl.pallas_call(
        _prep_kernel,
        in_specs=[
            pl.BlockSpec((B, 1, _NPAD), lambda: (0, 0, 0)),
            pl.BlockSpec((B, 2, _NPAD), lambda: (0, 0, 0)),
            pl.BlockSpec((B, 4, _NPAD), lambda: (0, 0, 0)),
            pl.BlockSpec((B, 4, _NPAD), lambda: (0, 0, 0)),
            pl.BlockSpec((B, 4), lambda: (0, 0)),
        ],
        out_specs=pl.BlockSpec((B, 6, _NPAD), lambda: (0, 0, 0)),
        out_shape=jax.ShapeDtypeStruct((B, 6, _NPAD), jnp.float32),
    )(lc, cls_t, ll, pb, lt)

    mesh = plsc.VectorSubcoreMesh(core_axis_name="c", subcore_axis_name="s",
                                  num_cores=_NC, num_subcores=_NS)
    parts = pl.kernel(
        _sc_pair_kernel,
        out_type=jax.ShapeDtypeStruct((_NC * _NS, 48), jnp.float32),
        mesh=mesh,
        scratch_types=[
            pltpu.VMEM((6, _NPAD), jnp.float32),
            pltpu.VMEM((48,), jnp.float32),
        ],
    )(pr)

    l1, l2 = pl.pallas_call(
        _finalize_kernel,
        in_specs=[
            pl.BlockSpec((_NC * _NS, 48), lambda: (0, 0)),
            pl.BlockSpec((B, 1, 1), lambda: (0, 0, 0)),
        ],
        out_specs=[
            pl.BlockSpec((1, 1), lambda: (0, 0), memory_space=pltpu.SMEM),
            pl.BlockSpec((1, 1), lambda: (0, 0), memory_space=pltpu.SMEM),
        ],
        out_shape=[
            jax.ShapeDtypeStruct((1, 1), jnp.float32),
            jax.ShapeDtypeStruct((1, 1), jnp.float32),
        ],
    )(parts, did)
    return (l1[0, 0], l2[0, 0])


# speedup vs baseline: 1.1230x; 1.1230x over previous
import functools

import jax
import jax.numpy as jnp
from jax import lax
from jax.experimental import pallas as pl
from jax.experimental.pallas import tpu as pltpu
from jax.experimental.pallas import tpu_sc as plsc

_G1 = 3.0
_G2 = 3.0
_PSHIFT = 15.0
_NPAD = 640
_NC = 2
_NS = 16
_CHUNKS = 4
_CVECS = _NPAD // (_CHUNKS * 16)
_TCS = 12
_SCS = _NS - _TCS
_NSL = (_NC * _NS) // _SCS
_AVSL = _NPAD // 16 // _NSL


def _prep_kernel(lc_ref, cls_ref, ll_ref, pb_ref, lt_ref, pr_ref):
    mask = lc_ref[:, 0, :] > 0
    p = jnp.exp(cls_ref[:, 1, :])

    bx1 = pb_ref[:, 0, :]
    by1 = pb_ref[:, 1, :]
    bx2 = pb_ref[:, 2, :]
    by2 = pb_ref[:, 3, :]
    tx1 = lt_ref[:, 0:1]
    ty1 = lt_ref[:, 1:2]
    tx2 = lt_ref[:, 2:3]
    ty2 = lt_ref[:, 3:4]

    xx1 = jnp.maximum(tx1, bx1)
    yy1 = jnp.maximum(ty1, by1)
    xx2 = jnp.minimum(tx2, bx2)
    yy2 = jnp.minimum(ty2, by2)
    ww = jnp.maximum(xx2 - xx1, 0.0)
    hh = jnp.maximum(yy2 - yy1, 0.0)
    area = (bx2 - bx1) * (by2 - by1)
    ta = (tx2 - tx1) * (ty2 - ty1)
    inter = ww * hh
    iou = inter / (area + ta - inter)

    cx = ll_ref[:, 0, :] + tx1
    cy = ll_ref[:, 1, :] + ty1
    tcx = (tx1 + tx2) / 2.0
    tcy = (ty1 + ty2) / 2.0
    dist = jnp.sqrt((cx - tcx) ** 2 + (cy - tcy) ** 2)

    ps = p - _PSHIFT
    pr_ref[:, 0, :] = jnp.where(mask, dist, -1e30)
    pr_ref[:, 1, :] = jnp.minimum(jnp.exp(_G1 * ps), 3e37)
    pr_ref[:, 2, :] = jnp.exp(_G2 * iou)
    pr_ref[:, 3, :] = jnp.where(mask, dist + 1.0, 1e30)
    pr_ref[:, 4, :] = jnp.where(mask, jnp.exp(-_G1 * ps), 0.0)
    pr_ref[:, 5, :] = jnp.where(mask, jnp.exp(-_G2 * iou), 0.0)


def _sc_pair_kernel(pr_hbm, out_hbm, buf, obuf):
    wid = lax.axis_index("c") * _NS + lax.axis_index("s")
    sample = _TCS + wid // _NSL
    aslice = wid % _NSL

    pltpu.sync_copy(pr_hbm.at[sample], buf)

    zero = jnp.zeros((16,), jnp.float32)

    abase = aslice * (_AVSL * 16)
    t1, t2, t3 = zero, zero, zero
    for c in range(_CHUNKS):
        bd = [buf[0, pl.ds(c * _CVECS * 16 + j * 16, 16)] for j in range(_CVECS)]
        bp = [buf[1, pl.ds(c * _CVECS * 16 + j * 16, 16)] for j in range(_CVECS)]
        bi = [buf[2, pl.ds(c * _CVECS * 16 + j * 16, 16)] for j in range(_CVECS)]

        def body(k, carry, bd=bd, bp=bp, bi=bi):
            t1, t2, t3 = carry
            a0 = abase + k * 16
            tav = buf[3, pl.ds(a0, 16)]
            e1pv = buf[4, pl.ds(a0, 16)]
            e1iv = buf[5, pl.ds(a0, 16)]
            for l in range(16):
                ta = tav[l]
                v1 = v2 = v3 = zero
                for j in range(_CVECS):
                    m = bd[j] >= ta
                    v1 = v1 + jnp.where(m, bp[j], 0.0)
                    v2 = v2 + jnp.where(m, bi[j], 0.0)
                    v3 = v3 + jnp.where(m, 1.0, 0.0)
                t1 = t1 + e1pv[l] * v1
                t2 = t2 + e1iv[l] * v2
                t3 = t3 + v3
            return (t1, t2, t3)

        t1, t2, t3 = lax.fori_loop(0, _AVSL, body, (t1, t2, t3))

    obuf[pl.ds(0, 16)] = t1
    obuf[pl.ds(16, 16)] = t2
    obuf[pl.ds(32, 16)] = t3
    pltpu.sync_copy(obuf, out_hbm.at[wid])


def _tc_pair_kernel(pr_ref, prt_ref, out_ref):
    db = pr_ref[0, 0:1, :]
    bp = pr_ref[0, 1:2, :]
    bi = pr_ref[0, 2:3, :]
    tcol = prt_ref[0, :, 0:1]
    mf = jnp.where(db >= tcol, 1.0, 0.0)

    v1 = jnp.sum(mf * bp, axis=1, keepdims=True)
    v2 = jnp.sum(mf * bi, axis=1, keepdims=True)
    v3 = jnp.sum(mf, axis=1, keepdims=True)

    e1p = prt_ref[0, :, 1:2]
    e1i = prt_ref[0, :, 2:3]
    out_ref[0, 0, 0] = jnp.sum(e1p * v1)
    out_ref[0, 0, 1] = jnp.sum(e1i * v2)
    out_ref[0, 0, 2] = jnp.sum(v3)


def _finalize_kernel(tc_ref, sc_ref, did_ref, l1_ref, l2_ref):
    s1_tc = tc_ref[:, 0, 0:1]
    s2_tc = tc_ref[:, 0, 1:2]
    cnt_tc = tc_ref[:, 0, 2:3]
    did_tc = did_ref[0:_TCS, 0, :]
    valid = (did_tc != 1) & (cnt_tc > 0.0)
    l1s = jnp.sum(jnp.where(valid, s1_tc / cnt_tc, 0.0))
    l2s = jnp.sum(jnp.where(valid, s2_tc / cnt_tc, 0.0))
    nv = jnp.sum(valid.astype(jnp.float32))
    for g in range(_SCS):
        blk = sc_ref[g * _NSL:(g + 1) * _NSL, :]
        row = jnp.sum(blk, axis=0, keepdims=True)
        s1 = jnp.sum(row[:, 0:16])
        s2 = jnp.sum(row[:, 16:32])
        cnt = jnp.sum(row[:, 32:48])
        vg = (did_ref[_TCS + g, 0, 0] != 1) & (cnt > 0.0)
        l1s = l1s + jnp.where(vg, s1 / cnt, 0.0)
        l2s = l2s + jnp.where(vg, s2 / cnt, 0.0)
        nv = nv + jnp.where(vg, 1.0, 0.0)
    l1_ref[0, 0] = jnp.where(nv > 0.0, l1s / nv, 0.0)
    l2_ref[0, 0] = jnp.where(nv > 0.0, l2s / nv, 0.0)


def kernel(cls, label_cls, label_loc, pred_bboxes, label_target, dataset_id):
    B = label_cls.shape[0]
    N = label_cls.shape[2] * label_cls.shape[3]
    assert B == _NS and N <= _NPAD
    pad = _NPAD - N

    lc = jnp.pad(jnp.reshape(label_cls, (B, 1, N)), ((0, 0), (0, 0), (0, pad)))
    cls_t = jnp.pad(jnp.transpose(jnp.reshape(cls, (B, N, 2)), (0, 2, 1)),
                    ((0, 0), (0, 0), (0, pad)))
    ll = jnp.pad(jnp.reshape(label_loc, (B, 4, N)), ((0, 0), (0, 0), (0, pad)))
    pb = jnp.pad(pred_bboxes, ((0, 0), (0, 0), (0, pad)))
    lt = jnp.reshape(label_target, (B, 4))
    did = jnp.reshape(dataset_id, (B, 1, 1))

    pr = pl.pallas_call(
        _prep_kernel,
        in_specs=[
            pl.BlockSpec((B, 1, _NPAD), lambda: (0, 0, 0)),
            pl.BlockSpec((B, 2, _NPAD), lambda: (0, 0, 0)),
            pl.BlockSpec((B, 4, _NPAD), lambda: (0, 0, 0)),
            pl.BlockSpec((B, 4, _NPAD), lambda: (0, 0, 0)),
            pl.BlockSpec((B, 4), lambda: (0, 0)),
        ],
        out_specs=pl.BlockSpec((B, 6, _NPAD), lambda: (0, 0, 0)),
        out_shape=jax.ShapeDtypeStruct((B, 6, _NPAD), jnp.float32),
    )(lc, cls_t, ll, pb, lt)

    prt = jnp.transpose(pr[:, 3:6, :], (0, 2, 1))

    mesh = plsc.VectorSubcoreMesh(core_axis_name="c", subcore_axis_name="s",
                                  num_cores=_NC, num_subcores=_NS)
    sc_parts = pl.kernel(
        _sc_pair_kernel,
        out_type=jax.ShapeDtypeStruct((_NC * _NS, 48), jnp.float32),
        mesh=mesh,
        scratch_types=[
            pltpu.VMEM((6, _NPAD), jnp.float32),
            pltpu.VMEM((48,), jnp.float32),
        ],
    )(pr)

    tc_parts = pl.pallas_call(
        _tc_pair_kernel,
        grid=(_TCS,),
        in_specs=[
            pl.BlockSpec((1, 6, _NPAD), lambda b: (b, 0, 0)),
            pl.BlockSpec((1, _NPAD, 3), lambda b: (b, 0, 0)),
        ],
        out_specs=pl.BlockSpec((1, 1, 8), lambda b: (b, 0, 0),
                               memory_space=pltpu.SMEM),
        out_shape=jax.ShapeDtypeStruct((_TCS, 1, 8), jnp.float32),
    )(pr, prt)

    l1, l2 = pl.pallas_call(
        _finalize_kernel,
        in_specs=[
            pl.BlockSpec((_TCS, 1, 8), lambda: (0, 0, 0)),
            pl.BlockSpec((_NC * _NS, 48), lambda: (0, 0)),
            pl.BlockSpec((B, 1, 1), lambda: (0, 0, 0)),
        ],
        out_specs=[
            pl.BlockSpec((1, 1), lambda: (0, 0), memory_space=pltpu.SMEM),
            pl.BlockSpec((1, 1), lambda: (0, 0), memory_space=pltpu.SMEM),
        ],
        out_shape=[
            jax.ShapeDtypeStruct((1, 1), jnp.float32),
            jax.ShapeDtypeStruct((1, 1), jnp.float32),
        ],
    )(tc_parts, sc_parts, did)
    return (l1[0, 0], l2[0, 0])


# speedup vs baseline: 1.2641x; 1.1257x over previous
import functools

import jax
import jax.numpy as jnp
from jax import lax
from jax.experimental import pallas as pl
from jax.experimental.pallas import tpu as pltpu
from jax.experimental.pallas import tpu_sc as plsc

_G1 = 3.0
_G2 = 3.0
_PSHIFT = 15.0
_NPAD = 640
_NC = 2
_NS = 16
_CHUNKS = 4
_CVECS = _NPAD // (_CHUNKS * 16)
_TCS = 12
_SCS = _NS - _TCS
_NSL = (_NC * _NS) // _SCS
_AVSL = _NPAD // 16 // _NSL


def _prep_kernel(lc_ref, cls_ref, ll_ref, pb_ref, lt_ref, pr_ref):
    mask = lc_ref[:, 0, :] > 0
    p = jnp.exp(cls_ref[:, 1, :])

    bx1 = pb_ref[:, 0, :]
    by1 = pb_ref[:, 1, :]
    bx2 = pb_ref[:, 2, :]
    by2 = pb_ref[:, 3, :]
    tx1 = lt_ref[:, 0:1]
    ty1 = lt_ref[:, 1:2]
    tx2 = lt_ref[:, 2:3]
    ty2 = lt_ref[:, 3:4]

    xx1 = jnp.maximum(tx1, bx1)
    yy1 = jnp.maximum(ty1, by1)
    xx2 = jnp.minimum(tx2, bx2)
    yy2 = jnp.minimum(ty2, by2)
    ww = jnp.maximum(xx2 - xx1, 0.0)
    hh = jnp.maximum(yy2 - yy1, 0.0)
    area = (bx2 - bx1) * (by2 - by1)
    ta = (tx2 - tx1) * (ty2 - ty1)
    inter = ww * hh
    iou = inter / (area + ta - inter)

    cx = ll_ref[:, 0, :] + tx1
    cy = ll_ref[:, 1, :] + ty1
    tcx = (tx1 + tx2) / 2.0
    tcy = (ty1 + ty2) / 2.0
    dist = jnp.sqrt((cx - tcx) ** 2 + (cy - tcy) ** 2)

    ps = p - _PSHIFT
    B = mask.shape[0]
    npad = _NPAD - mask.shape[1]

    def wr(row, x, padval):
        padcols = jnp.full((B, npad), padval, jnp.float32)
        pr_ref[:, row, :] = jnp.concatenate([x, padcols], axis=1)

    wr(0, jnp.where(mask, dist, -1e30), -1e30)
    wr(1, jnp.minimum(jnp.exp(_G1 * ps), 3e37), 0.0)
    wr(2, jnp.exp(_G2 * iou), 0.0)
    wr(3, jnp.where(mask, dist + 1.0, 1e30), 1e30)
    wr(4, jnp.where(mask, jnp.exp(-_G1 * ps), 0.0), 0.0)
    wr(5, jnp.where(mask, jnp.exp(-_G2 * iou), 0.0), 0.0)


def _sc_pair_kernel(pr_hbm, out_hbm, buf, obuf):
    wid = lax.axis_index("c") * _NS + lax.axis_index("s")
    sample = _TCS + wid // _NSL
    aslice = wid % _NSL

    pltpu.sync_copy(pr_hbm.at[sample], buf)

    zero = jnp.zeros((16,), jnp.float32)

    abase = aslice * (_AVSL * 16)
    t1, t2, t3 = zero, zero, zero
    for c in range(_CHUNKS):
        bd = [buf[0, pl.ds(c * _CVECS * 16 + j * 16, 16)] for j in range(_CVECS)]
        bp = [buf[1, pl.ds(c * _CVECS * 16 + j * 16, 16)] for j in range(_CVECS)]
        bi = [buf[2, pl.ds(c * _CVECS * 16 + j * 16, 16)] for j in range(_CVECS)]

        def body(k, carry, bd=bd, bp=bp, bi=bi):
            t1, t2, t3 = carry
            a0 = abase + k * 16
            tav = buf[3, pl.ds(a0, 16)]
            e1pv = buf[4, pl.ds(a0, 16)]
            e1iv = buf[5, pl.ds(a0, 16)]
            for l in range(16):
                ta = tav[l]
                v1 = v2 = v3 = zero
                for j in range(_CVECS):
                    m = bd[j] >= ta
                    v1 = v1 + jnp.where(m, bp[j], 0.0)
                    v2 = v2 + jnp.where(m, bi[j], 0.0)
                    v3 = v3 + jnp.where(m, 1.0, 0.0)
                t1 = t1 + e1pv[l] * v1
                t2 = t2 + e1iv[l] * v2
                t3 = t3 + v3
            return (t1, t2, t3)

        t1, t2, t3 = lax.fori_loop(0, _AVSL, body, (t1, t2, t3))

    obuf[pl.ds(0, 16)] = t1
    obuf[pl.ds(16, 16)] = t2
    obuf[pl.ds(32, 16)] = t3
    pltpu.sync_copy(obuf, out_hbm.at[wid])


def _tc_pair_kernel(pr_ref, out_ref):
    db = pr_ref[0, 0:1, :]
    bp = pr_ref[0, 1:2, :]
    bi = pr_ref[0, 2:3, :]
    tcol = jnp.transpose(pr_ref[0, 3:4, :], (1, 0))
    mf = jnp.where(db >= tcol, 1.0, 0.0)

    v1 = jnp.sum(mf * bp, axis=1, keepdims=True)
    v2 = jnp.sum(mf * bi, axis=1, keepdims=True)
    v3 = jnp.sum(mf, axis=1, keepdims=True)

    e1p = jnp.transpose(pr_ref[0, 4:5, :], (1, 0))
    e1i = jnp.transpose(pr_ref[0, 5:6, :], (1, 0))
    out_ref[0, 0, 0] = jnp.sum(e1p * v1)
    out_ref[0, 0, 1] = jnp.sum(e1i * v2)
    out_ref[0, 0, 2] = jnp.sum(v3)


def _finalize_kernel(tc_ref, sc_ref, did_ref, l1_ref, l2_ref):
    s1_tc = tc_ref[:, 0, 0:1]
    s2_tc = tc_ref[:, 0, 1:2]
    cnt_tc = tc_ref[:, 0, 2:3]
    did_tc = did_ref[0:_TCS, 0, :]
    valid = (did_tc != 1) & (cnt_tc > 0.0)
    l1s = jnp.sum(jnp.where(valid, s1_tc / cnt_tc, 0.0))
    l2s = jnp.sum(jnp.where(valid, s2_tc / cnt_tc, 0.0))
    nv = jnp.sum(valid.astype(jnp.float32))
    for g in range(_SCS):
        blk = sc_ref[g * _NSL:(g + 1) * _NSL, :]
        row = jnp.sum(blk, axis=0, keepdims=True)
        s1 = jnp.sum(row[:, 0:16])
        s2 = jnp.sum(row[:, 16:32])
        cnt = jnp.sum(row[:, 32:48])
        vg = (did_ref[_TCS + g, 0, 0] != 1) & (cnt > 0.0)
        l1s = l1s + jnp.where(vg, s1 / cnt, 0.0)
        l2s = l2s + jnp.where(vg, s2 / cnt, 0.0)
        nv = nv + jnp.where(vg, 1.0, 0.0)
    l1_ref[0, 0] = jnp.where(nv > 0.0, l1s / nv, 0.0)
    l2_ref[0, 0] = jnp.where(nv > 0.0, l2s / nv, 0.0)


def kernel(cls, label_cls, label_loc, pred_bboxes, label_target, dataset_id):
    B = label_cls.shape[0]
    N = label_cls.shape[2] * label_cls.shape[3]
    assert B == _NS and N <= _NPAD

    lc = jnp.reshape(label_cls, (B, 1, N))
    cls_t = jnp.transpose(jnp.reshape(cls, (B, N, 2)), (0, 2, 1))
    ll = jnp.reshape(label_loc, (B, 4, N))
    pb = pred_bboxes
    lt = jnp.reshape(label_target, (B, 4))
    did = jnp.reshape(dataset_id, (B, 1, 1))

    pr = pl.pallas_call(
        _prep_kernel,
        in_specs=[
            pl.BlockSpec((B, 1, N), lambda: (0, 0, 0)),
            pl.BlockSpec((B, 2, N), lambda: (0, 0, 0)),
            pl.BlockSpec((B, 4, N), lambda: (0, 0, 0)),
            pl.BlockSpec((B, 4, N), lambda: (0, 0, 0)),
            pl.BlockSpec((B, 4), lambda: (0, 0)),
        ],
        out_specs=pl.BlockSpec((B, 6, _NPAD), lambda: (0, 0, 0)),
        out_shape=jax.ShapeDtypeStruct((B, 6, _NPAD), jnp.float32),
    )(lc, cls_t, ll, pb, lt)

    mesh = plsc.VectorSubcoreMesh(core_axis_name="c", subcore_axis_name="s",
                                  num_cores=_NC, num_subcores=_NS)
    sc_parts = pl.kernel(
        _sc_pair_kernel,
        out_type=jax.ShapeDtypeStruct((_NC * _NS, 48), jnp.float32),
        mesh=mesh,
        scratch_types=[
            pltpu.VMEM((6, _NPAD), jnp.float32),
            pltpu.VMEM((48,), jnp.float32),
        ],
    )(pr)

    tc_parts = pl.pallas_call(
        _tc_pair_kernel,
        grid=(_TCS,),
        in_specs=[
            pl.BlockSpec((1, 6, _NPAD), lambda b: (b, 0, 0)),
        ],
        out_specs=pl.BlockSpec((1, 1, 8), lambda b: (b, 0, 0),
                               memory_space=pltpu.SMEM),
        out_shape=jax.ShapeDtypeStruct((_TCS, 1, 8), jnp.float32),
    )(pr)

    l1, l2 = pl.pallas_call(
        _finalize_kernel,
        in_specs=[
            pl.BlockSpec((_TCS, 1, 8), lambda: (0, 0, 0)),
            pl.BlockSpec((_NC * _NS, 48), lambda: (0, 0)),
            pl.BlockSpec((B, 1, 1), lambda: (0, 0, 0)),
        ],
        out_specs=[
            pl.BlockSpec((1, 1), lambda: (0, 0), memory_space=pltpu.SMEM),
            pl.BlockSpec((1, 1), lambda: (0, 0), memory_space=pltpu.SMEM),
        ],
        out_shape=[
            jax.ShapeDtypeStruct((1, 1), jnp.float32),
            jax.ShapeDtypeStruct((1, 1), jnp.float32),
        ],
    )(tc_parts, sc_parts, did)
    return (l1[0, 0], l2[0, 0])


# speedup vs baseline: 1.2743x; 1.0080x over previous
import functools

import jax
import jax.numpy as jnp
from jax import lax
from jax.experimental import pallas as pl
from jax.experimental.pallas import tpu as pltpu
from jax.experimental.pallas import tpu_sc as plsc

_G1 = 3.0
_G2 = 3.0
_PSHIFT = 15.0
_NPAD = 640
_NC = 2
_NS = 16
_CHUNKS = 4
_CVECS = _NPAD // (_CHUNKS * 16)
_TCS = 12
_SCS = _NS - _TCS
_NSL = (_NC * _NS) // _SCS
_AVSL = _NPAD // 16 // _NSL


def _prep_kernel(lc_ref, cls_ref, ll_ref, pb_ref, lt_ref, pr_ref):
    mask = lc_ref[:, 0, :] > 0
    p = jnp.exp(cls_ref[:, 1, :])

    bx1 = pb_ref[:, 0, :]
    by1 = pb_ref[:, 1, :]
    bx2 = pb_ref[:, 2, :]
    by2 = pb_ref[:, 3, :]
    tx1 = lt_ref[:, 0:1]
    ty1 = lt_ref[:, 1:2]
    tx2 = lt_ref[:, 2:3]
    ty2 = lt_ref[:, 3:4]

    xx1 = jnp.maximum(tx1, bx1)
    yy1 = jnp.maximum(ty1, by1)
    xx2 = jnp.minimum(tx2, bx2)
    yy2 = jnp.minimum(ty2, by2)
    ww = jnp.maximum(xx2 - xx1, 0.0)
    hh = jnp.maximum(yy2 - yy1, 0.0)
    area = (bx2 - bx1) * (by2 - by1)
    ta = (tx2 - tx1) * (ty2 - ty1)
    inter = ww * hh
    iou = inter / (area + ta - inter)

    cx = ll_ref[:, 0, :] + tx1
    cy = ll_ref[:, 1, :] + ty1
    tcx = (tx1 + tx2) / 2.0
    tcy = (ty1 + ty2) / 2.0
    dist = jnp.sqrt((cx - tcx) ** 2 + (cy - tcy) ** 2)

    ps = p - _PSHIFT
    B = mask.shape[0]
    npad = _NPAD - mask.shape[1]

    def wr(row, x, padval):
        padcols = jnp.full((B, npad), padval, jnp.float32)
        pr_ref[:, row, :] = jnp.concatenate([x, padcols], axis=1)

    wr(0, jnp.where(mask, dist, -1e30), -1e30)
    wr(1, jnp.minimum(jnp.exp(_G1 * ps), 3e37), 0.0)
    wr(2, jnp.exp(_G2 * iou), 0.0)
    wr(3, jnp.where(mask, dist + 1.0, 1e30), 1e30)
    wr(4, jnp.where(mask, jnp.exp(-_G1 * ps), 0.0), 0.0)
    wr(5, jnp.where(mask, jnp.exp(-_G2 * iou), 0.0), 0.0)


def _sc_pair_kernel(pr_hbm, out_hbm, buf, obuf):
    wid = lax.axis_index("c") * _NS + lax.axis_index("s")
    sample = _TCS + wid // _NSL
    aslice = wid % _NSL

    pltpu.sync_copy(pr_hbm.at[sample], buf)

    zero = jnp.zeros((16,), jnp.float32)

    abase = aslice * (_AVSL * 16)
    t1, t2, t3 = zero, zero, zero
    for c in range(_CHUNKS):
        bd = [buf[0, pl.ds(c * _CVECS * 16 + j * 16, 16)] for j in range(_CVECS)]
        bp = [buf[1, pl.ds(c * _CVECS * 16 + j * 16, 16)] for j in range(_CVECS)]
        bi = [buf[2, pl.ds(c * _CVECS * 16 + j * 16, 16)] for j in range(_CVECS)]

        def body(k, carry, bd=bd, bp=bp, bi=bi):
            t1, t2, t3 = carry
            a0 = abase + k * 16
            tav = buf[3, pl.ds(a0, 16)]
            e1pv = buf[4, pl.ds(a0, 16)]
            e1iv = buf[5, pl.ds(a0, 16)]
            for l in range(16):
                ta = tav[l]
                v1 = v2 = v3 = zero
                for j in range(_CVECS):
                    m = bd[j] >= ta
                    v1 = v1 + jnp.where(m, bp[j], 0.0)
                    v2 = v2 + jnp.where(m, bi[j], 0.0)
                    v3 = v3 + jnp.where(m, 1.0, 0.0)
                t1 = t1 + e1pv[l] * v1
                t2 = t2 + e1iv[l] * v2
                t3 = t3 + v3
            return (t1, t2, t3)

        t1, t2, t3 = lax.fori_loop(0, _AVSL, body, (t1, t2, t3))

    obuf[pl.ds(0, 16)] = t1
    obuf[pl.ds(16, 16)] = t2
    obuf[pl.ds(32, 16)] = t3
    pltpu.sync_copy(obuf, out_hbm.at[wid])


def _tc_pair_kernel(pr_ref, out_ref):
    db_c = jnp.transpose(pr_ref[0, 0:1, :], (1, 0))
    bp_c = jnp.transpose(pr_ref[0, 1:2, :], (1, 0))
    bi_c = jnp.transpose(pr_ref[0, 2:3, :], (1, 0))
    t_r = pr_ref[0, 3:4, :]
    mf = jnp.where(db_c >= t_r, 1.0, 0.0)

    v1 = jnp.sum(mf * bp_c, axis=0, keepdims=True)
    v2 = jnp.sum(mf * bi_c, axis=0, keepdims=True)
    v3 = jnp.sum(mf, axis=0, keepdims=True)

    e1p = pr_ref[0, 4:5, :]
    e1i = pr_ref[0, 5:6, :]
    out_ref[0, 0, 0] = jnp.sum(e1p * v1)
    out_ref[0, 0, 1] = jnp.sum(e1i * v2)
    out_ref[0, 0, 2] = jnp.sum(v3)


def _finalize_kernel(tc_ref, sc_ref, did_ref, l1_ref, l2_ref):
    s1_tc = tc_ref[:, 0, 0:1]
    s2_tc = tc_ref[:, 0, 1:2]
    cnt_tc = tc_ref[:, 0, 2:3]
    did_tc = did_ref[0:_TCS, 0, :]
    valid = (did_tc != 1) & (cnt_tc > 0.0)
    l1s = jnp.sum(jnp.where(valid, s1_tc / cnt_tc, 0.0))
    l2s = jnp.sum(jnp.where(valid, s2_tc / cnt_tc, 0.0))
    nv = jnp.sum(valid.astype(jnp.float32))
    for g in range(_SCS):
        blk = sc_ref[g * _NSL:(g + 1) * _NSL, :]
        row = jnp.sum(blk, axis=0, keepdims=True)
        s1 = jnp.sum(row[:, 0:16])
        s2 = jnp.sum(row[:, 16:32])
        cnt = jnp.sum(row[:, 32:48])
        vg = (did_ref[_TCS + g, 0, 0] != 1) & (cnt > 0.0)
        l1s = l1s + jnp.where(vg, s1 / cnt, 0.0)
        l2s = l2s + jnp.where(vg, s2 / cnt, 0.0)
        nv = nv + jnp.where(vg, 1.0, 0.0)
    l1_ref[0, 0] = jnp.where(nv > 0.0, l1s / nv, 0.0)
    l2_ref[0, 0] = jnp.where(nv > 0.0, l2s / nv, 0.0)


def kernel(cls, label_cls, label_loc, pred_bboxes, label_target, dataset_id):
    B = label_cls.shape[0]
    N = label_cls.shape[2] * label_cls.shape[3]
    assert B == _NS and N <= _NPAD

    lc = jnp.reshape(label_cls, (B, 1, N))
    cls_t = jnp.transpose(jnp.reshape(cls, (B, N, 2)), (0, 2, 1))
    ll = jnp.reshape(label_loc, (B, 4, N))
    pb = pred_bboxes
    lt = jnp.reshape(label_target, (B, 4))
    did = jnp.reshape(dataset_id, (B, 1, 1))

    pr = pl.pallas_call(
        _prep_kernel,
        in_specs=[
            pl.BlockSpec((B, 1, N), lambda: (0, 0, 0)),
            pl.BlockSpec((B, 2, N), lambda: (0, 0, 0)),
            pl.BlockSpec((B, 4, N), lambda: (0, 0, 0)),
            pl.BlockSpec((B, 4, N), lambda: (0, 0, 0)),
            pl.BlockSpec((B, 4), lambda: (0, 0)),
        ],
        out_specs=pl.BlockSpec((B, 6, _NPAD), lambda: (0, 0, 0)),
        out_shape=jax.ShapeDtypeStruct((B, 6, _NPAD), jnp.float32),
    )(lc, cls_t, ll, pb, lt)

    mesh = plsc.VectorSubcoreMesh(core_axis_name="c", subcore_axis_name="s",
                                  num_cores=_NC, num_subcores=_NS)
    sc_parts = pl.kernel(
        _sc_pair_kernel,
        out_type=jax.ShapeDtypeStruct((_NC * _NS, 48), jnp.float32),
        mesh=mesh,
        scratch_types=[
            pltpu.VMEM((6, _NPAD), jnp.float32),
            pltpu.VMEM((48,), jnp.float32),
        ],
    )(pr)

    tc_parts = pl.pallas_call(
        _tc_pair_kernel,
        grid=(_TCS,),
        in_specs=[
            pl.BlockSpec((1, 6, _NPAD), lambda b: (b, 0, 0)),
        ],
        out_specs=pl.BlockSpec((1, 1, 8), lambda b: (b, 0, 0),
                               memory_space=pltpu.SMEM),
        out_shape=jax.ShapeDtypeStruct((_TCS, 1, 8), jnp.float32),
    )(pr)

    l1, l2 = pl.pallas_call(
        _finalize_kernel,
        in_specs=[
            pl.BlockSpec((_TCS, 1, 8), lambda: (0, 0, 0)),
            pl.BlockSpec((_NC * _NS, 48), lambda: (0, 0)),
            pl.BlockSpec((B, 1, 1), lambda: (0, 0, 0)),
        ],
        out_specs=[
            pl.BlockSpec((1, 1), lambda: (0, 0), memory_space=pltpu.SMEM),
            pl.BlockSpec((1, 1), lambda: (0, 0), memory_space=pltpu.SMEM),
        ],
        out_shape=[
            jax.ShapeDtypeStruct((1, 1), jnp.float32),
            jax.ShapeDtypeStruct((1, 1), jnp.float32),
        ],
    )(tc_parts, sc_parts, did)
    return (l1[0, 0], l2[0, 0])
